# Initial kernel scaffold; baseline (speedup 1.0000x reference)
import jax
import jax.numpy as jnp
from jax.experimental import pallas as pl


def kernel(x, edge_index, edge_attr, time_weights, W_node, b_node, W_edge, b_edge, W_time, b_time, c1_w1, c1_b1, c1_g, c1_be, c1_w2, c1_b2, c2_w1, c2_b1, c2_g, c2_be, c2_w2, c2_b2, fc_w, fc_b):
    raise NotImplementedError("write your pallas kernel here")


# speedup vs baseline: 7.4760x; 7.4760x over previous
import functools

import jax
import jax.numpy as jnp
from jax import lax
from jax.experimental import pallas as pl
from jax.experimental.pallas import tpu as pltpu
from jax.experimental.pallas import tpu_sc as plsc

_NC = 2
_NS = 16
_NW = _NC * _NS
_CHUNK = 128


def _enc_nodes_body(x_ref, wn_ref, bn_ref, t_ref, wt_ref, bt_ref,
                    h_ref, hmax_ref):
    x0 = jnp.dot(x_ref[...], wn_ref[...],
                 preferred_element_type=jnp.float32) + bn_ref[...]
    tw = t_ref[...] * wt_ref[...] + bt_ref[...]
    h = x0 * tw
    h_ref[...] = h
    hmax_ref[...] = jnp.max(h, axis=0, keepdims=True)


def _enc_nodes(x, W_node, b_node, time_weights, W_time, b_time):
    n = x.shape[0]
    return pl.pallas_call(
        _enc_nodes_body,
        out_shape=[
            jax.ShapeDtypeStruct((n, 16), jnp.float32),
            jax.ShapeDtypeStruct((1, 16), jnp.float32),
        ],
    )(x, W_node, b_node.reshape(1, 16), time_weights,
      W_time, b_time.reshape(1, 16))


def _enc_edges_body(a_ref, w_ref, b_ref, ea_ref, emax_ref):
    i = pl.program_id(0)
    z = jnp.dot(a_ref[...], w_ref[...],
                preferred_element_type=jnp.float32) + b_ref[...]
    ea_ref[...] = z
    bm = jnp.max(z, axis=0, keepdims=True)

    @pl.when(i == 0)
    def _():
        emax_ref[...] = bm

    @pl.when(i > 0)
    def _():
        emax_ref[...] = jnp.maximum(emax_ref[...], bm)


def _enc_edges(edge_attr, W_edge, b_edge):
    e = edge_attr.shape[0]
    blk = 5000
    grid = e // blk
    return pl.pallas_call(
        _enc_edges_body,
        grid=(grid,),
        in_specs=[
            pl.BlockSpec((blk, 16), lambda i: (i, 0)),
            pl.BlockSpec((16, 16), lambda i: (0, 0)),
            pl.BlockSpec((1, 16), lambda i: (0, 0)),
        ],
        out_specs=[
            pl.BlockSpec((blk, 16), lambda i: (i, 0)),
            pl.BlockSpec((1, 16), lambda i: (0, 0)),
        ],
        out_shape=[
            jax.ShapeDtypeStruct((e, 16), jnp.float32),
            jax.ShapeDtypeStruct((1, 16), jnp.float32),
        ],
    )(edge_attr, W_edge, b_edge.reshape(1, 16))


def _edge_pass(h, ea, src, dst, shift):
    n = h.shape[0]
    e = ea.shape[0]
    n_chunks = e // _CHUNK
    iters = pl.cdiv(n_chunks, _NW)
    npad = ((n + 8 * _NS - 1) // (8 * _NS)) * (8 * _NS)
    zrows = npad // _NS

    mesh = plsc.VectorSubcoreMesh(core_axis_name="c", subcore_axis_name="s")

    @functools.partial(
        pl.kernel,
        out_type=jax.ShapeDtypeStruct((_NC * npad, 32), jnp.float32),
        mesh=mesh,
        scratch_types=[
            pltpu.VMEM((_CHUNK,), jnp.int32),
            pltpu.VMEM((_CHUNK,), jnp.int32),
            pltpu.VMEM((_CHUNK, 16), jnp.float32),
            pltpu.VMEM((_CHUNK, 16), jnp.float32),
            pltpu.VMEM((_CHUNK, 32), jnp.float32),
            pltpu.VMEM((16,), jnp.float32),
            pltpu.VMEM((zrows, 32), jnp.float32),
            pltpu.VMEM_SHARED((npad, 32), jnp.float32),
            pltpu.SemaphoreType.DMA,
        ],
        compiler_params=pltpu.CompilerParams(use_tc_tiling_on_sc=False),
    )
    def k(h_hbm, ea_hbm, src_hbm, dst_hbm, shift_hbm, out_hbm,
          sidx, didx, hrows, earows, tp, shv, zbuf, acc, sem):
        cid = lax.axis_index("c")
        sid = lax.axis_index("s")
        wid = sid * _NC + cid

        zero16 = jnp.zeros((16,), jnp.float32)

        def zrow(i, _):
            zbuf[i, pl.ds(0, 16)] = zero16
            zbuf[i, pl.ds(16, 16)] = zero16
            return 0

        lax.fori_loop(0, zrows, zrow, 0)
        pltpu.sync_copy(zbuf, acc.at[pl.ds(sid * zrows, zrows)])
        pltpu.sync_copy(shift_hbm, shv)
        shvec = shv[...]
        plsc.subcore_barrier()

        def body(i, _):
            c = i * _NW + wid

            @pl.when(c < n_chunks)
            def _():
                base = c * _CHUNK
                pltpu.sync_copy(src_hbm.at[pl.ds(base, _CHUNK)], sidx)
                pltpu.sync_copy(dst_hbm.at[pl.ds(base, _CHUNK)], didx)
                pltpu.async_copy(h_hbm.at[sidx], hrows, sem).wait()
                pltpu.sync_copy(ea_hbm.at[pl.ds(base, _CHUNK)], earows)

                def row(r, _):
                    msg = jnp.maximum(hrows[r, :] + earows[r, :], 0.0) + 1e-7
                    t = jnp.exp(msg - shvec)
                    tp[r, pl.ds(0, 16)] = t
                    tp[r, pl.ds(16, 16)] = t * msg
                    return 0

                lax.fori_loop(0, _CHUNK, row, 0)
                pltpu.sync_copy(tp, acc.at[didx], add=True)

            return 0

        lax.fori_loop(0, iters, body, 0)
        plsc.subcore_barrier()

        pltpu.sync_copy(acc.at[pl.ds(sid * zrows, zrows)], zbuf)
        pltpu.sync_copy(zbuf,
                        out_hbm.at[pl.ds(cid * npad + sid * zrows, zrows)])

    out = k(h, ea, src, dst, shift)
    return out.reshape(_NC, npad, 32)[:, :n, :]


def _layer_tail(sp_ref, h_ref, w1_ref, b1_ref, g_ref, be_ref, w2_ref, b2_ref):
    sp = sp_ref[0] + sp_ref[1]
    s = sp[:, :16]
    p = sp[:, 16:]
    den = jnp.where(s > 0, s, 1.0)
    aggr = jnp.where(s > 0, p / den, 0.0)
    hmid = h_ref[...] + aggr
    z = jnp.dot(hmid, w1_ref[...],
                preferred_element_type=jnp.float32) + b1_ref[...]
    mu = jnp.mean(z, axis=0, keepdims=True)
    var = jnp.mean((z - mu) ** 2, axis=0, keepdims=True)
    zn = (z - mu) * lax.rsqrt(var + 1e-5) * g_ref[...] + be_ref[...]
    zn = jnp.maximum(zn, 0.0)
    h2 = jnp.dot(zn, w2_ref[...],
                 preferred_element_type=jnp.float32) + b2_ref[...]
    return jnp.maximum(h2, 0.0)


def _combine1_body(sp_ref, h_ref, w1_ref, b1_ref, g_ref, be_ref,
                   w2_ref, b2_ref, out_ref, hmax_ref):
    h2 = _layer_tail(sp_ref, h_ref, w1_ref, b1_ref, g_ref, be_ref,
                     w2_ref, b2_ref)
    out_ref[...] = h2
    hmax_ref[...] = jnp.max(h2, axis=0, keepdims=True)


def _combine1(sp, h, w1, b1, g, be, w2, b2):
    n = h.shape[0]
    return pl.pallas_call(
        _combine1_body,
        out_shape=[
            jax.ShapeDtypeStruct((n, 16), jnp.float32),
            jax.ShapeDtypeStruct((1, 16), jnp.float32),
        ],
    )(sp, h, w1, b1.reshape(1, 32), g.reshape(1, 32), be.reshape(1, 32),
      w2, b2.reshape(1, 16))


def _combine2_body(sp_ref, h_ref, w1_ref, b1_ref, g_ref, be_ref,
                   w2_ref, b2_ref, fw_ref, fb_ref, out_ref):
    h2 = _layer_tail(sp_ref, h_ref, w1_ref, b1_ref, g_ref, be_ref,
                     w2_ref, b2_ref)
    logits = jnp.dot(h2, fw_ref[...],
                     preferred_element_type=jnp.float32) + fb_ref[...]
    mx = jnp.max(logits, axis=1, keepdims=True)
    lse = jnp.log(jnp.sum(jnp.exp(logits - mx), axis=1, keepdims=True)) + mx
    out_ref[...] = logits - lse


def _combine2(sp, h, w1, b1, g, be, w2, b2, fc_w, fc_b):
    n = h.shape[0]
    c = fc_w.shape[1]
    return pl.pallas_call(
        _combine2_body,
        out_shape=jax.ShapeDtypeStruct((n, c), jnp.float32),
    )(sp, h, w1, b1.reshape(1, 32), g.reshape(1, 32), be.reshape(1, 32),
      w2, b2.reshape(1, 16), fc_w, fc_b.reshape(1, c))


def kernel(x, edge_index, edge_attr, time_weights, W_node, b_node, W_edge,
           b_edge, W_time, b_time, c1_w1, c1_b1, c1_g, c1_be, c1_w2, c1_b2,
           c2_w1, c2_b1, c2_g, c2_be, c2_w2, c2_b2, fc_w, fc_b):
    src = edge_index[0]
    dst = edge_index[1]

    h0, hmax0 = _enc_nodes(x, W_node, b_node, time_weights, W_time, b_time)
    ea, eamax = _enc_edges(edge_attr, W_edge, b_edge)

    shift0 = (jnp.maximum(hmax0[0] + eamax[0], 0.0) + 1e-7)
    sp0 = _edge_pass(h0, ea, src, dst, shift0)
    h1, hmax1 = _combine1(sp0, h0, c1_w1, c1_b1, c1_g, c1_be, c1_w2, c1_b2)

    shift1 = (jnp.maximum(hmax1[0] + eamax[0], 0.0) + 1e-7)
    sp1 = _edge_pass(h1, ea, src, dst, shift1)
    return _combine2(sp1, h1, c2_w1, c2_b1, c2_g, c2_be, c2_w2, c2_b2,
                     fc_w, fc_b)


# speedup vs baseline: 16.1507x; 2.1603x over previous
import functools

import jax
import jax.numpy as jnp
from jax import lax
from jax.experimental import pallas as pl
from jax.experimental.pallas import tpu as pltpu
from jax.experimental.pallas import tpu_sc as plsc

_NC = 2
_NS = 16
_NW = _NC * _NS
_CHUNK = 128


def _enc_nodes_body(x_ref, wn_ref, bn_ref, t_ref, wt_ref, bt_ref,
                    h_ref, hmax_ref):
    x0 = jnp.dot(x_ref[...], wn_ref[...],
                 preferred_element_type=jnp.float32) + bn_ref[...]
    tw = t_ref[...] * wt_ref[...] + bt_ref[...]
    h = x0 * tw
    h_ref[...] = h
    hmax_ref[...] = jnp.max(h, axis=0, keepdims=True)


def _enc_nodes(x, W_node, b_node, time_weights, W_time, b_time):
    n = x.shape[0]
    return pl.pallas_call(
        _enc_nodes_body,
        out_shape=[
            jax.ShapeDtypeStruct((n, 16), jnp.float32),
            jax.ShapeDtypeStruct((1, 16), jnp.float32),
        ],
    )(x, W_node, b_node.reshape(1, 16), time_weights,
      W_time, b_time.reshape(1, 16))


def _enc_edges_body(a_ref, w_ref, b_ref, ea_ref, emax_ref):
    i = pl.program_id(0)
    z = jnp.dot(a_ref[...], w_ref[...],
                preferred_element_type=jnp.float32) + b_ref[...]
    ea_ref[...] = z
    bm = jnp.max(z, axis=0, keepdims=True)

    @pl.when(i == 0)
    def _():
        emax_ref[...] = bm

    @pl.when(i > 0)
    def _():
        emax_ref[...] = jnp.maximum(emax_ref[...], bm)


def _enc_edges(edge_attr, W_edge, b_edge):
    e = edge_attr.shape[0]
    blk = 5000
    grid = e // blk
    return pl.pallas_call(
        _enc_edges_body,
        grid=(grid,),
        in_specs=[
            pl.BlockSpec((blk, 16), lambda i: (i, 0)),
            pl.BlockSpec((16, 16), lambda i: (0, 0)),
            pl.BlockSpec((1, 16), lambda i: (0, 0)),
        ],
        out_specs=[
            pl.BlockSpec((blk, 16), lambda i: (i, 0)),
            pl.BlockSpec((1, 16), lambda i: (0, 0)),
        ],
        out_shape=[
            jax.ShapeDtypeStruct((e, 16), jnp.float32),
            jax.ShapeDtypeStruct((1, 16), jnp.float32),
        ],
    )(edge_attr, W_edge, b_edge.reshape(1, 16))


_G = 4
_GE = _G * _CHUNK


def _edge_pass(h, ea, src, dst, shift):
    n = h.shape[0]
    e = ea.shape[0]
    n_chunks = e // _CHUNK
    n_groups = n_chunks // _G
    iters = pl.cdiv(n_groups, _NW)
    pairs = pl.cdiv(iters, 2)
    npad = ((n + 8 * _NS - 1) // (8 * _NS)) * (8 * _NS)
    zrows = npad // _NS

    src2 = src.reshape(n_chunks, _CHUNK)
    dst2 = dst.reshape(n_chunks, _CHUNK)

    mesh = plsc.VectorSubcoreMesh(core_axis_name="c", subcore_axis_name="s")

    @functools.partial(
        pl.kernel,
        out_type=jax.ShapeDtypeStruct((_NC * npad, 32), jnp.float32),
        mesh=mesh,
        scratch_types=[
            pltpu.VMEM((2, _G, _CHUNK), jnp.int32),
            pltpu.VMEM((2, _G, _CHUNK), jnp.int32),
            pltpu.VMEM((2, _GE, 16), jnp.float32),
            pltpu.VMEM((2, _GE, 16), jnp.float32),
            pltpu.VMEM((2, _GE, 32), jnp.float32),
            pltpu.VMEM((16,), jnp.float32),
            pltpu.VMEM((zrows, 32), jnp.float32),
            pltpu.VMEM_SHARED((npad, 32), jnp.float32),
            pltpu.SemaphoreType.DMA,
            pltpu.SemaphoreType.DMA,
            pltpu.SemaphoreType.DMA,
            pltpu.SemaphoreType.DMA,
            pltpu.SemaphoreType.DMA,
        ],
        compiler_params=pltpu.CompilerParams(use_tc_tiling_on_sc=False),
    )
    def k(h_hbm, ea_hbm, src_hbm, dst_hbm, shift_hbm, out_hbm,
          sidx, didx, hrows, earows, tp, shv, zbuf, acc,
          sem_s, sem_d, sem_e, sem_g, sem_sc):
        cid = lax.axis_index("c")
        sid = lax.axis_index("s")
        wid = sid * _NC + cid

        zero16 = jnp.zeros((16,), jnp.float32)

        @plsc.parallel_loop(0, zrows, step=1, unroll=4)
        def _(i):
            zbuf[i, pl.ds(0, 16)] = zero16
            zbuf[i, pl.ds(16, 16)] = zero16

        pltpu.sync_copy(zbuf, acc.at[pl.ds(sid * zrows, zrows)])
        pltpu.sync_copy(shift_hbm, shv)
        shvec = shv[...]
        plsc.subcore_barrier()

        def issue_loads(g, q):
            pltpu.async_copy(src_hbm.at[pl.ds(g * _G, _G)], sidx.at[q], sem_s)
            pltpu.async_copy(dst_hbm.at[pl.ds(g * _G, _G)], didx.at[q], sem_d)
            pltpu.async_copy(ea_hbm.at[pl.ds(g * _GE, _GE)], earows.at[q],
                             sem_e)

        def drain_loads(g, q):
            pltpu.make_async_copy(src_hbm.at[pl.ds(g * _G, _G)], sidx.at[q],
                                  sem_s).wait()
            pltpu.make_async_copy(dst_hbm.at[pl.ds(g * _G, _G)], didx.at[q],
                                  sem_d).wait()

        issue_loads(wid, 0)

        def pair_body(j, _):
            for q in (0, 1):
                i = j * 2 + q
                g = i * _NW + wid
                g_next = g + _NW

                @pl.when(g < n_groups)
                def _():
                    drain_loads(g, q)
                    for b in range(_G):
                        pltpu.async_copy(
                            h_hbm.at[sidx.at[q, b]],
                            hrows.at[q, pl.ds(b * _CHUNK, _CHUNK)], sem_g)

                @pl.when(g_next < n_groups)
                def _():
                    issue_loads(g_next, 1 - q)

                @pl.when(g < n_groups)
                def _():
                    pltpu.make_async_copy(
                        ea_hbm.at[pl.ds(g * _GE, _GE)], earows.at[q],
                        sem_e).wait()
                    for b in range(_G):
                        pltpu.make_async_copy(
                            h_hbm.at[sidx.at[q, b]],
                            hrows.at[q, pl.ds(b * _CHUNK, _CHUNK)],
                            sem_g).wait()

                    @plsc.parallel_loop(0, _GE, step=1, unroll=4)
                    def _(r):
                        msg = jnp.maximum(hrows[q, r, :] + earows[q, r, :],
                                          0.0) + 1e-7
                        t = jnp.exp(msg - shvec)
                        tp[q, r, pl.ds(0, 16)] = t
                        tp[q, r, pl.ds(16, 16)] = t * msg

                    for b in range(_G):
                        pltpu.async_copy(
                            tp.at[q, pl.ds(b * _CHUNK, _CHUNK)],
                            acc.at[didx.at[q, b]], sem_sc, add=True)
                    for b in range(_G):
                        pltpu.make_async_copy(
                            tp.at[q, pl.ds(b * _CHUNK, _CHUNK)],
                            acc.at[didx.at[q, b]], sem_sc).wait()

            return 0

        lax.fori_loop(0, pairs, pair_body, 0)
        plsc.subcore_barrier()

        pltpu.sync_copy(acc.at[pl.ds(sid * zrows, zrows)], zbuf)
        pltpu.sync_copy(zbuf,
                        out_hbm.at[pl.ds(cid * npad + sid * zrows, zrows)])

    out = k(h, ea, src2, dst2, shift)
    return out.reshape(_NC, npad, 32)[:, :n, :]


def _layer_tail(sp_ref, h_ref, w1_ref, b1_ref, g_ref, be_ref, w2_ref, b2_ref):
    sp = sp_ref[0] + sp_ref[1]
    s = sp[:, :16]
    p = sp[:, 16:]
    den = jnp.where(s > 0, s, 1.0)
    aggr = jnp.where(s > 0, p / den, 0.0)
    hmid = h_ref[...] + aggr
    z = jnp.dot(hmid, w1_ref[...],
                preferred_element_type=jnp.float32) + b1_ref[...]
    mu = jnp.mean(z, axis=0, keepdims=True)
    var = jnp.mean((z - mu) ** 2, axis=0, keepdims=True)
    zn = (z - mu) * lax.rsqrt(var + 1e-5) * g_ref[...] + be_ref[...]
    zn = jnp.maximum(zn, 0.0)
    h2 = jnp.dot(zn, w2_ref[...],
                 preferred_element_type=jnp.float32) + b2_ref[...]
    return jnp.maximum(h2, 0.0)


def _combine1_body(sp_ref, h_ref, w1_ref, b1_ref, g_ref, be_ref,
                   w2_ref, b2_ref, out_ref, hmax_ref):
    h2 = _layer_tail(sp_ref, h_ref, w1_ref, b1_ref, g_ref, be_ref,
                     w2_ref, b2_ref)
    out_ref[...] = h2
    hmax_ref[...] = jnp.max(h2, axis=0, keepdims=True)


def _combine1(sp, h, w1, b1, g, be, w2, b2):
    n = h.shape[0]
    return pl.pallas_call(
        _combine1_body,
        out_shape=[
            jax.ShapeDtypeStruct((n, 16), jnp.float32),
            jax.ShapeDtypeStruct((1, 16), jnp.float32),
        ],
    )(sp, h, w1, b1.reshape(1, 32), g.reshape(1, 32), be.reshape(1, 32),
      w2, b2.reshape(1, 16))


def _combine2_body(sp_ref, h_ref, w1_ref, b1_ref, g_ref, be_ref,
                   w2_ref, b2_ref, fw_ref, fb_ref, out_ref):
    h2 = _layer_tail(sp_ref, h_ref, w1_ref, b1_ref, g_ref, be_ref,
                     w2_ref, b2_ref)
    logits = jnp.dot(h2, fw_ref[...],
                     preferred_element_type=jnp.float32) + fb_ref[...]
    mx = jnp.max(logits, axis=1, keepdims=True)
    lse = jnp.log(jnp.sum(jnp.exp(logits - mx), axis=1, keepdims=True)) + mx
    out_ref[...] = logits - lse


def _combine2(sp, h, w1, b1, g, be, w2, b2, fc_w, fc_b):
    n = h.shape[0]
    c = fc_w.shape[1]
    return pl.pallas_call(
        _combine2_body,
        out_shape=jax.ShapeDtypeStruct((n, c), jnp.float32),
    )(sp, h, w1, b1.reshape(1, 32), g.reshape(1, 32), be.reshape(1, 32),
      w2, b2.reshape(1, 16), fc_w, fc_b.reshape(1, c))


def kernel(x, edge_index, edge_attr, time_weights, W_node, b_node, W_edge,
           b_edge, W_time, b_time, c1_w1, c1_b1, c1_g, c1_be, c1_w2, c1_b2,
           c2_w1, c2_b1, c2_g, c2_be, c2_w2, c2_b2, fc_w, fc_b):
    src = edge_index[0]
    dst = edge_index[1]

    h0, hmax0 = _enc_nodes(x, W_node, b_node, time_weights, W_time, b_time)
    ea, eamax = _enc_edges(edge_attr, W_edge, b_edge)

    shift0 = (jnp.maximum(hmax0[0] + eamax[0], 0.0) + 1e-7)
    sp0 = _edge_pass(h0, ea, src, dst, shift0)
    h1, hmax1 = _combine1(sp0, h0, c1_w1, c1_b1, c1_g, c1_be, c1_w2, c1_b2)

    shift1 = (jnp.maximum(hmax1[0] + eamax[0], 0.0) + 1e-7)
    sp1 = _edge_pass(h1, ea, src, dst, shift1)
    return _combine2(sp1, h1, c2_w1, c2_b1, c2_g, c2_be, c2_w2, c2_b2,
                     fc_w, fc_b)


# speedup vs baseline: 24.4647x; 1.5148x over previous
import functools

import jax
import jax.numpy as jnp
from jax import lax
from jax.experimental import pallas as pl
from jax.experimental.pallas import tpu as pltpu
from jax.experimental.pallas import tpu_sc as plsc

_NC = 2
_NS = 16
_NW = _NC * _NS
_CHUNK = 128


def _enc_nodes_body(x_ref, wn_ref, bn_ref, t_ref, wt_ref, bt_ref,
                    h_ref, hmax_ref):
    x0 = jnp.dot(x_ref[...], wn_ref[...],
                 preferred_element_type=jnp.float32) + bn_ref[...]
    tw = t_ref[...] * wt_ref[...] + bt_ref[...]
    h = x0 * tw
    h_ref[...] = h
    hmax_ref[...] = jnp.max(h, axis=0, keepdims=True)


def _enc_nodes(x, W_node, b_node, time_weights, W_time, b_time):
    n = x.shape[0]
    return pl.pallas_call(
        _enc_nodes_body,
        out_shape=[
            jax.ShapeDtypeStruct((n, 16), jnp.float32),
            jax.ShapeDtypeStruct((1, 16), jnp.float32),
        ],
    )(x, W_node, b_node.reshape(1, 16), time_weights,
      W_time, b_time.reshape(1, 16))


def _enc_edges_body(a_ref, w_ref, b_ref, ea_ref, emax_ref):
    i = pl.program_id(0)
    z = jnp.dot(a_ref[...], w_ref[...],
                preferred_element_type=jnp.float32) + b_ref[...]
    ea_ref[...] = z
    bm = jnp.max(z, axis=0, keepdims=True)

    @pl.when(i == 0)
    def _():
        emax_ref[...] = bm

    @pl.when(i > 0)
    def _():
        emax_ref[...] = jnp.maximum(emax_ref[...], bm)


def _enc_edges(edge_attr, W_edge, b_edge):
    e = edge_attr.shape[0]
    ep = e // 8
    attr_p = edge_attr.reshape(ep, 128)
    w_kron = jnp.kron(jnp.eye(8, dtype=jnp.float32), W_edge)
    b_tile = jnp.tile(b_edge, 8).reshape(1, 128)
    blk = 5000
    grid = ep // blk
    ea_p, emax_p = pl.pallas_call(
        _enc_edges_body,
        grid=(grid,),
        in_specs=[
            pl.BlockSpec((blk, 128), lambda i: (i, 0)),
            pl.BlockSpec((128, 128), lambda i: (0, 0)),
            pl.BlockSpec((1, 128), lambda i: (0, 0)),
        ],
        out_specs=[
            pl.BlockSpec((blk, 128), lambda i: (i, 0)),
            pl.BlockSpec((1, 128), lambda i: (0, 0)),
        ],
        out_shape=[
            jax.ShapeDtypeStruct((ep, 128), jnp.float32),
            jax.ShapeDtypeStruct((1, 128), jnp.float32),
        ],
    )(attr_p, w_kron, b_tile)
    return ea_p, emax_p


_G = 4
_GE = _G * _CHUNK


def _edge_pass(h, ea_p, src, dst, hmax, eamax_p):
    n = h.shape[0]
    e = ea_p.shape[0] * 8
    ea = ea_p.reshape(e, 16)
    n_chunks = e // _CHUNK
    n_groups = n_chunks // _G
    iters = pl.cdiv(n_groups, _NW)
    pairs = pl.cdiv(iters, 2)
    npad = ((n + 8 * _NS - 1) // (8 * _NS)) * (8 * _NS)
    zrows = npad // _NS

    src2 = src.reshape(n_chunks, _CHUNK)
    dst2 = dst.reshape(n_chunks, _CHUNK)

    mesh = plsc.VectorSubcoreMesh(core_axis_name="c", subcore_axis_name="s")

    @functools.partial(
        pl.kernel,
        out_type=jax.ShapeDtypeStruct((_NC * npad, 32), jnp.float32),
        mesh=mesh,
        scratch_types=[
            pltpu.VMEM((2, _G, _CHUNK), jnp.int32),
            pltpu.VMEM((2, _G, _CHUNK), jnp.int32),
            pltpu.VMEM((2, _GE, 16), jnp.float32),
            pltpu.VMEM((2, _GE, 16), jnp.float32),
            pltpu.VMEM((2, _GE, 32), jnp.float32),
            pltpu.VMEM((16,), jnp.float32),
            pltpu.VMEM((8, 16), jnp.float32),
            pltpu.VMEM((zrows, 32), jnp.float32),
            pltpu.VMEM_SHARED((npad, 32), jnp.float32),
            pltpu.SemaphoreType.DMA,
            pltpu.SemaphoreType.DMA,
            pltpu.SemaphoreType.DMA,
            pltpu.SemaphoreType.DMA,
            pltpu.SemaphoreType.DMA,
        ],
        compiler_params=pltpu.CompilerParams(use_tc_tiling_on_sc=False),
    )
    def k(h_hbm, ea_hbm, src_hbm, dst_hbm, hmax_hbm, eamax_hbm, out_hbm,
          sidx, didx, hrows, earows, tp, hmv, emv, zbuf, acc,
          sem_s, sem_d, sem_e, sem_g, sem_sc):
        cid = lax.axis_index("c")
        sid = lax.axis_index("s")
        wid = sid * _NC + cid

        zero16 = jnp.zeros((16,), jnp.float32)

        @plsc.parallel_loop(0, zrows, step=1, unroll=4)
        def _(i):
            zbuf[i, pl.ds(0, 16)] = zero16
            zbuf[i, pl.ds(16, 16)] = zero16

        pltpu.sync_copy(zbuf, acc.at[pl.ds(sid * zrows, zrows)])
        pltpu.sync_copy(hmax_hbm, hmv)
        pltpu.sync_copy(eamax_hbm, emv)
        em = emv[0, :]
        for kk in range(1, 8):
            em = jnp.maximum(em, emv[kk, :])
        shvec = jnp.maximum(hmv[...] + em, 0.0) + 1e-7
        plsc.subcore_barrier()

        def issue_loads(g, q):
            pltpu.async_copy(src_hbm.at[pl.ds(g * _G, _G)], sidx.at[q], sem_s)
            pltpu.async_copy(dst_hbm.at[pl.ds(g * _G, _G)], didx.at[q], sem_d)
            pltpu.async_copy(ea_hbm.at[pl.ds(g * _GE, _GE)], earows.at[q],
                             sem_e)

        def drain_loads(g, q):
            pltpu.make_async_copy(src_hbm.at[pl.ds(g * _G, _G)], sidx.at[q],
                                  sem_s).wait()
            pltpu.make_async_copy(dst_hbm.at[pl.ds(g * _G, _G)], didx.at[q],
                                  sem_d).wait()

        issue_loads(wid, 0)

        def pair_body(j, _):
            for q in (0, 1):
                i = j * 2 + q
                g = i * _NW + wid
                g_next = g + _NW

                @pl.when(g < n_groups)
                def _():
                    drain_loads(g, q)
                    for b in range(_G):
                        pltpu.async_copy(
                            h_hbm.at[sidx.at[q, b]],
                            hrows.at[q, pl.ds(b * _CHUNK, _CHUNK)], sem_g)

                @pl.when(g_next < n_groups)
                def _():
                    issue_loads(g_next, 1 - q)

                @pl.when(g < n_groups)
                def _():
                    pltpu.make_async_copy(
                        ea_hbm.at[pl.ds(g * _GE, _GE)], earows.at[q],
                        sem_e).wait()
                    for b in range(_G):
                        pltpu.make_async_copy(
                            h_hbm.at[sidx.at[q, b]],
                            hrows.at[q, pl.ds(b * _CHUNK, _CHUNK)],
                            sem_g).wait()

                    @plsc.parallel_loop(0, _GE, step=1, unroll=4)
                    def _(r):
                        msg = jnp.maximum(hrows[q, r, :] + earows[q, r, :],
                                          0.0) + 1e-7
                        t = jnp.exp(msg - shvec)
                        tp[q, r, pl.ds(0, 16)] = t
                        tp[q, r, pl.ds(16, 16)] = t * msg

                    for b in range(_G):
                        pltpu.async_copy(
                            tp.at[q, pl.ds(b * _CHUNK, _CHUNK)],
                            acc.at[didx.at[q, b]], sem_sc, add=True)
                    for b in range(_G):
                        pltpu.make_async_copy(
                            tp.at[q, pl.ds(b * _CHUNK, _CHUNK)],
                            acc.at[didx.at[q, b]], sem_sc).wait()

            return 0

        lax.fori_loop(0, pairs, pair_body, 0)
        plsc.subcore_barrier()

        pltpu.sync_copy(acc.at[pl.ds(sid * zrows, zrows)], zbuf)
        pltpu.sync_copy(zbuf,
                        out_hbm.at[pl.ds(cid * npad + sid * zrows, zrows)])

    out = k(h, ea, src2, dst2, hmax.reshape(16), eamax_p.reshape(8, 16))
    return out.reshape(_NC, npad, 32)


def _layer_tail(sp_ref, h_ref, w1_ref, b1_ref, g_ref, be_ref, w2_ref, b2_ref):
    n = h_ref.shape[0]
    sp = sp_ref[0, :n, :] + sp_ref[1, :n, :]
    s = sp[:, :16]
    p = sp[:, 16:]
    den = jnp.where(s > 0, s, 1.0)
    aggr = jnp.where(s > 0, p / den, 0.0)
    hmid = h_ref[...] + aggr
    z = jnp.dot(hmid, w1_ref[...],
                preferred_element_type=jnp.float32) + b1_ref[...]
    mu = jnp.mean(z, axis=0, keepdims=True)
    var = jnp.mean((z - mu) ** 2, axis=0, keepdims=True)
    zn = (z - mu) * lax.rsqrt(var + 1e-5) * g_ref[...] + be_ref[...]
    zn = jnp.maximum(zn, 0.0)
    h2 = jnp.dot(zn, w2_ref[...],
                 preferred_element_type=jnp.float32) + b2_ref[...]
    return jnp.maximum(h2, 0.0)


def _combine1_body(sp_ref, h_ref, w1_ref, b1_ref, g_ref, be_ref,
                   w2_ref, b2_ref, out_ref, hmax_ref):
    h2 = _layer_tail(sp_ref, h_ref, w1_ref, b1_ref, g_ref, be_ref,
                     w2_ref, b2_ref)
    out_ref[...] = h2
    hmax_ref[...] = jnp.max(h2, axis=0, keepdims=True)


def _combine1(sp, h, w1, b1, g, be, w2, b2):
    n = h.shape[0]
    return pl.pallas_call(
        _combine1_body,
        out_shape=[
            jax.ShapeDtypeStruct((n, 16), jnp.float32),
            jax.ShapeDtypeStruct((1, 16), jnp.float32),
        ],
    )(sp, h, w1, b1.reshape(1, 32), g.reshape(1, 32), be.reshape(1, 32),
      w2, b2.reshape(1, 16))


def _combine2_body(sp_ref, h_ref, w1_ref, b1_ref, g_ref, be_ref,
                   w2_ref, b2_ref, fw_ref, fb_ref, out_ref):
    h2 = _layer_tail(sp_ref, h_ref, w1_ref, b1_ref, g_ref, be_ref,
                     w2_ref, b2_ref)
    logits = jnp.dot(h2, fw_ref[...],
                     preferred_element_type=jnp.float32) + fb_ref[...]
    mx = jnp.max(logits, axis=1, keepdims=True)
    lse = jnp.log(jnp.sum(jnp.exp(logits - mx), axis=1, keepdims=True)) + mx
    out_ref[...] = logits - lse


def _combine2(sp, h, w1, b1, g, be, w2, b2, fc_w, fc_b):
    n = h.shape[0]
    c = fc_w.shape[1]
    return pl.pallas_call(
        _combine2_body,
        out_shape=jax.ShapeDtypeStruct((n, c), jnp.float32),
    )(sp, h, w1, b1.reshape(1, 32), g.reshape(1, 32), be.reshape(1, 32),
      w2, b2.reshape(1, 16), fc_w, fc_b.reshape(1, c))


def kernel(x, edge_index, edge_attr, time_weights, W_node, b_node, W_edge,
           b_edge, W_time, b_time, c1_w1, c1_b1, c1_g, c1_be, c1_w2, c1_b2,
           c2_w1, c2_b1, c2_g, c2_be, c2_w2, c2_b2, fc_w, fc_b):
    src = edge_index[0]
    dst = edge_index[1]

    h0, hmax0 = _enc_nodes(x, W_node, b_node, time_weights, W_time, b_time)
    ea, eamax = _enc_edges(edge_attr, W_edge, b_edge)

    sp0 = _edge_pass(h0, ea, src, dst, hmax0, eamax)
    h1, hmax1 = _combine1(sp0, h0, c1_w1, c1_b1, c1_g, c1_be, c1_w2, c1_b2)

    sp1 = _edge_pass(h1, ea, src, dst, hmax1, eamax)
    return _combine2(sp1, h1, c2_w1, c2_b1, c2_g, c2_be, c2_w2, c2_b2,
                     fc_w, fc_b)


# speedup vs baseline: 26.7058x; 1.0916x over previous
import functools

import jax
import jax.numpy as jnp
from jax import lax
from jax.experimental import pallas as pl
from jax.experimental.pallas import tpu as pltpu
from jax.experimental.pallas import tpu_sc as plsc

_NC = 2
_NS = 16
_NW = _NC * _NS
_CHUNK = 128


def _enc_nodes_body(x_ref, wn_ref, bn_ref, t_ref, wt_ref, bt_ref,
                    h_ref, hmax_ref):
    x0 = jnp.dot(x_ref[...], wn_ref[...],
                 preferred_element_type=jnp.float32) + bn_ref[...]
    tw = t_ref[...] * wt_ref[...] + bt_ref[...]
    h = x0 * tw
    h_ref[...] = h
    hmax_ref[...] = jnp.max(h, axis=0, keepdims=True)


def _enc_nodes(x, W_node, b_node, time_weights, W_time, b_time):
    n = x.shape[0]
    return pl.pallas_call(
        _enc_nodes_body,
        out_shape=[
            jax.ShapeDtypeStruct((n, 16), jnp.float32),
            jax.ShapeDtypeStruct((1, 16), jnp.float32),
        ],
    )(x, W_node, b_node.reshape(1, 16), time_weights,
      W_time, b_time.reshape(1, 16))


def _enc_edges_body(a_ref, w_ref, b_ref, ea_ref, emax_ref):
    i = pl.program_id(0)
    z = jnp.dot(a_ref[...], w_ref[...],
                preferred_element_type=jnp.float32) + b_ref[...]
    ea_ref[...] = z
    bm = jnp.max(z, axis=0, keepdims=True)

    @pl.when(i == 0)
    def _():
        emax_ref[...] = bm

    @pl.when(i > 0)
    def _():
        emax_ref[...] = jnp.maximum(emax_ref[...], bm)


def _enc_edges(edge_attr, W_edge, b_edge):
    e = edge_attr.shape[0]
    ep = e // 8
    attr_p = edge_attr.reshape(ep, 128)
    w_kron = jnp.kron(jnp.eye(8, dtype=jnp.float32), W_edge)
    b_tile = jnp.tile(b_edge, 8).reshape(1, 128)
    blk = 5000
    grid = ep // blk
    ea_p, emax_p = pl.pallas_call(
        _enc_edges_body,
        grid=(grid,),
        in_specs=[
            pl.BlockSpec((blk, 128), lambda i: (i, 0)),
            pl.BlockSpec((128, 128), lambda i: (0, 0)),
            pl.BlockSpec((1, 128), lambda i: (0, 0)),
        ],
        out_specs=[
            pl.BlockSpec((blk, 128), lambda i: (i, 0)),
            pl.BlockSpec((1, 128), lambda i: (0, 0)),
        ],
        out_shape=[
            jax.ShapeDtypeStruct((ep, 128), jnp.float32),
            jax.ShapeDtypeStruct((1, 128), jnp.float32),
        ],
    )(attr_p, w_kron, b_tile)
    return ea_p, emax_p


_G = 5
_GE = _G * _CHUNK


def _edge_pass(h, ea_p, src, dst, hmax, eamax_p):
    n = h.shape[0]
    e = ea_p.shape[0] * 8
    ea = ea_p.reshape(e, 16)
    n_chunks = e // _CHUNK
    n_groups = n_chunks // _G
    iters = pl.cdiv(n_groups, _NW)
    pairs = pl.cdiv(iters, 2)
    npad = ((n + 8 * _NS - 1) // (8 * _NS)) * (8 * _NS)
    zrows = npad // _NS

    src2 = src.reshape(n_chunks, _CHUNK)
    dst2 = dst.reshape(n_chunks, _CHUNK)

    mesh = plsc.VectorSubcoreMesh(core_axis_name="c", subcore_axis_name="s")

    @functools.partial(
        pl.kernel,
        out_type=jax.ShapeDtypeStruct((_NC * npad, 32), jnp.float32),
        mesh=mesh,
        scratch_types=[
            pltpu.VMEM((2, _G, _CHUNK), jnp.int32),
            pltpu.VMEM((2, _G, _CHUNK), jnp.int32),
            pltpu.VMEM((2, _GE, 16), jnp.float32),
            pltpu.VMEM((2, _GE, 16), jnp.float32),
            pltpu.VMEM((2, _GE, 32), jnp.float32),
            pltpu.VMEM((16,), jnp.float32),
            pltpu.VMEM((8, 16), jnp.float32),
            pltpu.VMEM((zrows, 32), jnp.float32),
            pltpu.VMEM_SHARED((npad, 32), jnp.float32),
            pltpu.SemaphoreType.DMA,
            pltpu.SemaphoreType.DMA,
            pltpu.SemaphoreType.DMA,
            pltpu.SemaphoreType.DMA,
            pltpu.SemaphoreType.DMA,
        ],
        compiler_params=pltpu.CompilerParams(use_tc_tiling_on_sc=False),
    )
    def k(h_hbm, ea_hbm, src_hbm, dst_hbm, hmax_hbm, eamax_hbm, out_hbm,
          sidx, didx, hrows, earows, tp, hmv, emv, zbuf, acc,
          sem_s, sem_d, sem_e, sem_g, sem_sc):
        cid = lax.axis_index("c")
        sid = lax.axis_index("s")
        wid = sid * _NC + cid

        zero16 = jnp.zeros((16,), jnp.float32)

        @plsc.parallel_loop(0, zrows, step=1, unroll=4)
        def _(i):
            zbuf[i, pl.ds(0, 16)] = zero16
            zbuf[i, pl.ds(16, 16)] = zero16

        pltpu.sync_copy(zbuf, acc.at[pl.ds(sid * zrows, zrows)])
        pltpu.sync_copy(hmax_hbm, hmv)
        pltpu.sync_copy(eamax_hbm, emv)
        em = emv[0, :]
        for kk in range(1, 8):
            em = jnp.maximum(em, emv[kk, :])
        shvec = jnp.maximum(hmv[...] + em, 0.0) + 1e-7
        plsc.subcore_barrier()

        def issue_loads(g, q):
            pltpu.async_copy(src_hbm.at[pl.ds(g * _G, _G)], sidx.at[q], sem_s)
            pltpu.async_copy(dst_hbm.at[pl.ds(g * _G, _G)], didx.at[q], sem_d)
            pltpu.async_copy(ea_hbm.at[pl.ds(g * _GE, _GE)], earows.at[q],
                             sem_e)

        def drain_loads(g, q):
            pltpu.make_async_copy(src_hbm.at[pl.ds(g * _G, _G)], sidx.at[q],
                                  sem_s).wait()
            pltpu.make_async_copy(dst_hbm.at[pl.ds(g * _G, _G)], didx.at[q],
                                  sem_d).wait()

        issue_loads(wid, 0)

        def drain_scatters(q):
            for b in range(_G):
                pltpu.make_async_copy(
                    tp.at[q, pl.ds(b * _CHUNK, _CHUNK)],
                    acc.at[didx.at[q, b]], sem_sc).wait()

        def pair_body(j, _):
            for q in (0, 1):
                i = j * 2 + q
                g = i * _NW + wid
                g_next = g + _NW

                @pl.when(g < n_groups)
                def _():
                    drain_loads(g, q)
                    for b in range(_G):
                        pltpu.async_copy(
                            h_hbm.at[sidx.at[q, b]],
                            hrows.at[q, pl.ds(b * _CHUNK, _CHUNK)], sem_g)

                @pl.when((i >= 1) & (g - _NW < n_groups))
                def _():
                    drain_scatters(1 - q)

                @pl.when(g_next < n_groups)
                def _():
                    issue_loads(g_next, 1 - q)

                @pl.when(g < n_groups)
                def _():
                    pltpu.make_async_copy(
                        ea_hbm.at[pl.ds(g * _GE, _GE)], earows.at[q],
                        sem_e).wait()
                    for b in range(_G):
                        pltpu.make_async_copy(
                            h_hbm.at[sidx.at[q, b]],
                            hrows.at[q, pl.ds(b * _CHUNK, _CHUNK)],
                            sem_g).wait()

                    @plsc.parallel_loop(0, _GE, step=1, unroll=4)
                    def _(r):
                        msg = jnp.maximum(hrows[q, r, :] + earows[q, r, :],
                                          0.0) + 1e-7
                        t = jnp.exp(msg - shvec)
                        tp[q, r, pl.ds(0, 16)] = t
                        tp[q, r, pl.ds(16, 16)] = t * msg

                    for b in range(_G):
                        pltpu.async_copy(
                            tp.at[q, pl.ds(b * _CHUNK, _CHUNK)],
                            acc.at[didx.at[q, b]], sem_sc, add=True)

            return 0

        lax.fori_loop(0, pairs, pair_body, 0)

        g_last = (iters - 1) * _NW + wid

        @pl.when(g_last < n_groups)
        def _():
            drain_scatters((iters - 1) % 2)

        plsc.subcore_barrier()

        pltpu.sync_copy(acc.at[pl.ds(sid * zrows, zrows)], zbuf)
        pltpu.sync_copy(zbuf,
                        out_hbm.at[pl.ds(cid * npad + sid * zrows, zrows)])

    out = k(h, ea, src2, dst2, hmax.reshape(16), eamax_p.reshape(8, 16))
    return out.reshape(_NC, npad, 32)


def _layer_tail(sp_ref, h_ref, w1_ref, b1_ref, g_ref, be_ref, w2_ref, b2_ref):
    n = h_ref.shape[0]
    sp = sp_ref[0, :n, :] + sp_ref[1, :n, :]
    s = sp[:, :16]
    p = sp[:, 16:]
    den = jnp.where(s > 0, s, 1.0)
    aggr = jnp.where(s > 0, p / den, 0.0)
    hmid = h_ref[...] + aggr
    z = jnp.dot(hmid, w1_ref[...],
                preferred_element_type=jnp.float32) + b1_ref[...]
    mu = jnp.mean(z, axis=0, keepdims=True)
    var = jnp.mean((z - mu) ** 2, axis=0, keepdims=True)
    zn = (z - mu) * lax.rsqrt(var + 1e-5) * g_ref[...] + be_ref[...]
    zn = jnp.maximum(zn, 0.0)
    h2 = jnp.dot(zn, w2_ref[...],
                 preferred_element_type=jnp.float32) + b2_ref[...]
    return jnp.maximum(h2, 0.0)


def _combine1_body(sp_ref, h_ref, w1_ref, b1_ref, g_ref, be_ref,
                   w2_ref, b2_ref, out_ref, hmax_ref):
    h2 = _layer_tail(sp_ref, h_ref, w1_ref, b1_ref, g_ref, be_ref,
                     w2_ref, b2_ref)
    out_ref[...] = h2
    hmax_ref[...] = jnp.max(h2, axis=0, keepdims=True)


def _combine1(sp, h, w1, b1, g, be, w2, b2):
    n = h.shape[0]
    return pl.pallas_call(
        _combine1_body,
        out_shape=[
            jax.ShapeDtypeStruct((n, 16), jnp.float32),
            jax.ShapeDtypeStruct((1, 16), jnp.float32),
        ],
    )(sp, h, w1, b1.reshape(1, 32), g.reshape(1, 32), be.reshape(1, 32),
      w2, b2.reshape(1, 16))


def _combine2_body(sp_ref, h_ref, w1_ref, b1_ref, g_ref, be_ref,
                   w2_ref, b2_ref, fw_ref, fb_ref, out_ref):
    h2 = _layer_tail(sp_ref, h_ref, w1_ref, b1_ref, g_ref, be_ref,
                     w2_ref, b2_ref)
    logits = jnp.dot(h2, fw_ref[...],
                     preferred_element_type=jnp.float32) + fb_ref[...]
    mx = jnp.max(logits, axis=1, keepdims=True)
    lse = jnp.log(jnp.sum(jnp.exp(logits - mx), axis=1, keepdims=True)) + mx
    out_ref[...] = logits - lse


def _combine2(sp, h, w1, b1, g, be, w2, b2, fc_w, fc_b):
    n = h.shape[0]
    c = fc_w.shape[1]
    return pl.pallas_call(
        _combine2_body,
        out_shape=jax.ShapeDtypeStruct((n, c), jnp.float32),
    )(sp, h, w1, b1.reshape(1, 32), g.reshape(1, 32), be.reshape(1, 32),
      w2, b2.reshape(1, 16), fc_w, fc_b.reshape(1, c))


def kernel(x, edge_index, edge_attr, time_weights, W_node, b_node, W_edge,
           b_edge, W_time, b_time, c1_w1, c1_b1, c1_g, c1_be, c1_w2, c1_b2,
           c2_w1, c2_b1, c2_g, c2_be, c2_w2, c2_b2, fc_w, fc_b):
    src = edge_index[0]
    dst = edge_index[1]

    h0, hmax0 = _enc_nodes(x, W_node, b_node, time_weights, W_time, b_time)
    ea, eamax = _enc_edges(edge_attr, W_edge, b_edge)

    sp0 = _edge_pass(h0, ea, src, dst, hmax0, eamax)
    h1, hmax1 = _combine1(sp0, h0, c1_w1, c1_b1, c1_g, c1_be, c1_w2, c1_b2)

    sp1 = _edge_pass(h1, ea, src, dst, hmax1, eamax)
    return _combine2(sp1, h1, c2_w1, c2_b1, c2_g, c2_be, c2_w2, c2_b2,
                     fc_w, fc_b)
